# Initial kernel scaffold; baseline (speedup 1.0000x reference)
import jax
import jax.numpy as jnp
from jax.experimental import pallas as pl


def kernel(x, in_degree, out_degree, z_in, z_out):
    raise NotImplementedError("write your pallas kernel here")


# speedup vs baseline: 2.3804x; 2.3804x over previous
import functools

import jax
import jax.numpy as jnp
from jax import lax
from jax.experimental import pallas as pl
from jax.experimental.pallas import tpu as pltpu
from jax.experimental.pallas import tpu_sc as plsc

_N = 100000
_D = 128
_K = 128
_FULL = _N // _K
_TAIL = _N - _FULL * _K
_NW = 32
_TAIL_WID = 13
_LANES = 16


def _chunk_compute(a, b, acc, nrows):
    def row(r, carry):
        for v in range(_D // _LANES):
            sl = pl.ds(v * _LANES, _LANES)
            av = a[r, sl]
            bv = b[r, sl]
            plsc.addupdate(acc.at[r, sl], av + bv)
        return carry

    lax.fori_loop(0, nrows, row, 0, unroll=False)


def _sc_body(x_hbm, din_hbm, dout_hbm, zin_hbm, zout_hbm, out_hbm,
             ii, io, acc, a, b, s1, s2):
    wid = lax.axis_index("s") * 2 + lax.axis_index("c")

    def do_chunk(base, nrows):
        rows = pl.ds(base, nrows)
        pltpu.sync_copy(din_hbm.at[rows], ii.at[pl.ds(0, nrows)])
        pltpu.sync_copy(dout_hbm.at[rows], io.at[pl.ds(0, nrows)])
        g1 = pltpu.async_copy(zin_hbm.at[ii.at[pl.ds(0, nrows)]],
                              a.at[pl.ds(0, nrows)], s1)
        g2 = pltpu.async_copy(zout_hbm.at[io.at[pl.ds(0, nrows)]],
                              b.at[pl.ds(0, nrows)], s2)
        pltpu.sync_copy(x_hbm.at[rows], acc.at[pl.ds(0, nrows)])
        g1.wait()
        g2.wait()
        _chunk_compute(a, b, acc, nrows)
        pltpu.sync_copy(acc.at[pl.ds(0, nrows)], out_hbm.at[rows])

    nloops = (_FULL + _NW - 1) // _NW

    def body(t, carry):
        c = wid + t * _NW

        @pl.when(c < _FULL)
        def _():
            do_chunk(c * _K, _K)

        return carry

    lax.fori_loop(0, nloops, body, 0, unroll=False)

    @pl.when(wid == _TAIL_WID)
    def _():
        do_chunk(_FULL * _K, _TAIL)


@jax.jit
def _centrality(x2, din, dout, z_in, z_out):
    mesh = plsc.VectorSubcoreMesh(core_axis_name="c", subcore_axis_name="s")
    fn = functools.partial(
        pl.kernel,
        mesh=mesh,
        out_type=jax.ShapeDtypeStruct((_N, _D), jnp.float32),
        scratch_types=[
            pltpu.VMEM((_K,), jnp.int32),
            pltpu.VMEM((_K,), jnp.int32),
            pltpu.VMEM((_K, _D), jnp.float32),
            pltpu.VMEM((_K, _D), jnp.float32),
            pltpu.VMEM((_K, _D), jnp.float32),
            pltpu.SemaphoreType.DMA,
            pltpu.SemaphoreType.DMA,
        ],
    )(_sc_body)
    return fn(x2, din, dout, z_in, z_out)


def kernel(x, in_degree, out_degree, z_in, z_out):
    x2 = x.reshape(_N, _D)
    out2 = _centrality(x2, in_degree.astype(jnp.int32),
                       out_degree.astype(jnp.int32), z_in, z_out)
    return out2.reshape(x.shape)
